# baseline (speedup 1.0000x reference)
import functools

import jax
import jax.numpy as jnp
from jax import lax
from jax.experimental import pallas as pl
from jax.experimental.pallas import tpu as pltpu
from jax.experimental.pallas import tpu_sc as plsc

B = 16384
D = 32
BLK = 2048


@functools.cache
def _build_gather():
    info = plsc.get_sparse_core_info()
    nc, ns = info.num_cores, info.num_subcores
    nw = nc * ns
    bpw = B // nw
    mesh = plsc.VectorSubcoreMesh(core_axis_name="c", subcore_axis_name="s")

    @functools.partial(
        pl.kernel,
        mesh=mesh,
        compiler_params=pltpu.CompilerParams(use_tc_tiling_on_sc=False),
        out_type=[
            jax.ShapeDtypeStruct((B, D), jnp.float32),
            jax.ShapeDtypeStruct((B, D), jnp.float32),
        ],
        scratch_types=[
            pltpu.VMEM((bpw,), jnp.int32),
            pltpu.VMEM((bpw,), jnp.int32),
            pltpu.VMEM((bpw, D), jnp.float32),
            pltpu.VMEM((bpw, D), jnp.float32),
            pltpu.SemaphoreType.DMA,
            pltpu.SemaphoreType.DMA,
        ],
    )
    def gather(u_hbm, v_hbm, ut_hbm, vt_hbm, eu_hbm, ev_hbm,
               ui_v, vi_v, eu_v, ev_v, sem_u, sem_v):
        wid = lax.axis_index("s") * nc + lax.axis_index("c")
        base = wid * bpw
        pltpu.sync_copy(u_hbm.at[pl.ds(base, bpw)], ui_v)
        pltpu.sync_copy(v_hbm.at[pl.ds(base, bpw)], vi_v)
        cp_u = pltpu.async_copy(ut_hbm.at[ui_v], eu_v, sem_u)
        cp_v = pltpu.async_copy(vt_hbm.at[vi_v], ev_v, sem_v)
        cp_u.wait()
        cp_v.wait()
        pltpu.sync_copy(eu_v, eu_hbm.at[pl.ds(base, bpw)])
        pltpu.sync_copy(ev_v, ev_hbm.at[pl.ds(base, bpw)])

    return gather


def _mlp_body(eu_ref, ev_ref, w1u_ref, w1v_ref, b1_ref, w2t_ref, b2_ref,
              whg_ref, whh_ref, bh_ref, out_ref):
    eu = eu_ref[...]
    ev = ev_ref[...]
    h1 = jnp.dot(eu, w1u_ref[...], preferred_element_type=jnp.float32)
    h1 = h1 + jnp.dot(ev, w1v_ref[...], preferred_element_type=jnp.float32)
    h1 = jnp.maximum(h1 + b1_ref[...], 0.0)
    h2 = jnp.dot(h1, w2t_ref[...], preferred_element_type=jnp.float32)
    h2 = jnp.maximum(h2 + b2_ref[...], 0.0)
    gmf = eu * ev
    logit = jnp.dot(gmf, whg_ref[...], preferred_element_type=jnp.float32)
    logit = logit + jnp.dot(h2, whh_ref[...], preferred_element_type=jnp.float32)
    logit = logit + bh_ref[...]
    out_ref[...] = jax.nn.sigmoid(logit)


def _mlp_call(eu, ev, w1u, w1v, b1, w2t, b2, whg, whh, bh2d):
    grid = B // BLK
    full = lambda i: (0, 0)
    return pl.pallas_call(
        _mlp_body,
        grid=(grid,),
        in_specs=[
            pl.BlockSpec((BLK, D), lambda i: (i, 0)),
            pl.BlockSpec((BLK, D), lambda i: (i, 0)),
            pl.BlockSpec((D, 128), full),
            pl.BlockSpec((D, 128), full),
            pl.BlockSpec((1, 128), full),
            pl.BlockSpec((128, 32), full),
            pl.BlockSpec((1, 32), full),
            pl.BlockSpec((D, 1), full),
            pl.BlockSpec((32, 1), full),
            pl.BlockSpec((1, 1), full),
        ],
        out_specs=pl.BlockSpec((BLK, 1), lambda i: (i, 0)),
        out_shape=jax.ShapeDtypeStruct((B, 1), jnp.float32),
    )(eu, ev, w1u, w1v, b1, w2t, b2, whg, whh, bh2d)


def kernel(u, v, U_gmf, V_gmf, W1, b1, W2, b2, Wh, bh):
    eu, ev = _build_gather()(u.astype(jnp.int32), v.astype(jnp.int32),
                             U_gmf, V_gmf)
    w1u = W1[:, :D].T
    w1v = W1[:, D:].T
    w2t = W2.T
    whg = Wh[0, :D].reshape(D, 1)
    whh = Wh[0, D:].reshape(32, 1)
    return _mlp_call(eu, ev, w1u, w1v, b1.reshape(1, 128), w2t,
                     b2.reshape(1, 32), whg, whh, bh.reshape(1, 1))
